# baseline (speedup 1.0000x reference)
import functools

import jax
import jax.numpy as jnp
from jax import lax
from jax.experimental import pallas as pl
from jax.experimental.pallas import tpu as pltpu
from jax.experimental.pallas import tpu_sc as plsc

VOCAB = 73
VOCAB_PAD = 80
GLYPH_DIM = 256
OUT_DIM = 512
B, L = 1024, 200
NTOK = B * L
NWORKERS = 32
PER_W = NTOK // NWORKERS
CHUNK = 32
NCHUNK = PER_W // CHUNK


def _table_body(s_ref, w_ref, b_ref, o_ref):
    o_ref[0] = lax.dot_general(
        s_ref[...], w_ref[...],
        (((1,), (1,)), ((), ())),
        preferred_element_type=jnp.float32,
    ) + b_ref[...]


NBUF = 5
NREP = 5


def _gather_body(table_hbm, idx_hbm, out_hbm, idx_v, *bufs):
    rows = list(bufs[:NBUF])
    gsem = list(bufs[NBUF:2 * NBUF])
    osem = list(bufs[2 * NBUF:])
    cid = lax.axis_index("c")
    sid = lax.axis_index("s")
    wid = sid * 2 + cid
    row_base = wid * PER_W

    pltpu.sync_copy(idx_hbm.at[wid], idx_v)


    def out_slc(c):
        return out_hbm.at[pl.ds(row_base + c * CHUNK, CHUNK)]

    for k in range(NBUF):
        pltpu.async_copy(table_hbm.at[wid * NREP + k].at[idx_v.at[k]], rows[k], gsem[k])

    def step(s, _):
        for k in range(NBUF):
            c = s * NBUF + k
            pltpu.make_async_copy(table_hbm.at[wid * NREP + k].at[idx_v.at[c]],
                                  rows[k], gsem[k]).wait()
            pltpu.async_copy(rows[k], out_slc(c), osem[k])
        for k in range(NBUF):
            c = s * NBUF + k
            pltpu.make_async_copy(rows[k], out_slc(c), osem[k]).wait()

            @pl.when(c + NBUF < NCHUNK)
            def _():
                pltpu.async_copy(table_hbm.at[wid * NREP + k].at[idx_v.at[c + NBUF]],
                                 rows[k], gsem[k])
        return 0

    lax.fori_loop(0, NCHUNK // NBUF, step, 0)


def kernel(QR, syms, W, b):
    syms_pad = jnp.pad(syms, ((0, VOCAB_PAD - VOCAB), (0, 0)))
    table = pl.pallas_call(
        _table_body,
        grid=(NWORKERS * NREP,),
        in_specs=[
            pl.BlockSpec((VOCAB_PAD, GLYPH_DIM), lambda i: (0, 0)),
            pl.BlockSpec((OUT_DIM, GLYPH_DIM), lambda i: (0, 0)),
            pl.BlockSpec((1, OUT_DIM), lambda i: (0, 0)),
        ],
        out_specs=pl.BlockSpec((1, VOCAB_PAD, OUT_DIM), lambda i: (i, 0, 0)),
        out_shape=jax.ShapeDtypeStruct((NWORKERS * NREP, VOCAB_PAD, OUT_DIM),
                                       jnp.float32),
    )(syms_pad, W, b.reshape(1, OUT_DIM))

    idx = QR.reshape(NWORKERS, NCHUNK, CHUNK)
    mesh = plsc.VectorSubcoreMesh(core_axis_name="c", subcore_axis_name="s")
    gather = functools.partial(
        pl.kernel,
        out_type=jax.ShapeDtypeStruct((NTOK, OUT_DIM), jnp.float32),
        mesh=mesh,
        scratch_types=(
            [pltpu.VMEM((NCHUNK, CHUNK), jnp.int32)]
            + [pltpu.VMEM((CHUNK, OUT_DIM), jnp.float32)] * NBUF
            + [pltpu.SemaphoreType.DMA] * (2 * NBUF)
        ),
    )(_gather_body)
    out = gather(table, idx)
    return out.reshape(B, L, OUT_DIM)


# speedup vs baseline: 1.1153x; 1.1153x over previous
import functools

import jax
import jax.numpy as jnp
from jax import lax
from jax.experimental import pallas as pl
from jax.experimental.pallas import tpu as pltpu
from jax.experimental.pallas import tpu_sc as plsc

VOCAB = 73
VOCAB_PAD = 80
GLYPH_DIM = 256
OUT_DIM = 512
B, L = 1024, 200
NTOK = B * L
NWORKERS = 32
PER_W = NTOK // NWORKERS
CHUNK = 32
NCHUNK = PER_W // CHUNK


def _table_body(s_ref, w_ref, b_ref, o_ref):
    o_ref[0] = lax.dot_general(
        s_ref[...], w_ref[...],
        (((1,), (1,)), ((), ())),
        preferred_element_type=jnp.float32,
    ) + b_ref[...]


NBUF = 5
NREP = 2


def _gather_body(table_hbm, idx_hbm, out_hbm, idx_v, *bufs):
    rows = list(bufs[:NBUF])
    gsem = list(bufs[NBUF:2 * NBUF])
    osem = list(bufs[2 * NBUF:])
    cid = lax.axis_index("c")
    sid = lax.axis_index("s")
    wid = sid * 2 + cid
    row_base = wid * PER_W

    pltpu.sync_copy(idx_hbm.at[wid], idx_v)


    def out_slc(c):
        return out_hbm.at[pl.ds(row_base + c * CHUNK, CHUNK)]

    for k in range(NBUF):
        pltpu.async_copy(table_hbm.at[wid * NREP + (k % NREP)].at[idx_v.at[k]], rows[k], gsem[k])

    def step(s, _):
        for k in range(NBUF):
            c = s * NBUF + k
            pltpu.make_async_copy(table_hbm.at[wid * NREP + (k % NREP)].at[idx_v.at[c]],
                                  rows[k], gsem[k]).wait()
            pltpu.async_copy(rows[k], out_slc(c), osem[k])
        for k in range(NBUF):
            c = s * NBUF + k
            pltpu.make_async_copy(rows[k], out_slc(c), osem[k]).wait()

            @pl.when(c + NBUF < NCHUNK)
            def _():
                pltpu.async_copy(table_hbm.at[wid * NREP + (k % NREP)].at[idx_v.at[c + NBUF]],
                                 rows[k], gsem[k])
        return 0

    lax.fori_loop(0, NCHUNK // NBUF, step, 0)


def kernel(QR, syms, W, b):
    syms_pad = jnp.pad(syms, ((0, VOCAB_PAD - VOCAB), (0, 0)))
    table = pl.pallas_call(
        _table_body,
        grid=(NWORKERS * NREP,),
        in_specs=[
            pl.BlockSpec((VOCAB_PAD, GLYPH_DIM), lambda i: (0, 0)),
            pl.BlockSpec((OUT_DIM, GLYPH_DIM), lambda i: (0, 0)),
            pl.BlockSpec((1, OUT_DIM), lambda i: (0, 0)),
        ],
        out_specs=pl.BlockSpec((1, VOCAB_PAD, OUT_DIM), lambda i: (i, 0, 0)),
        out_shape=jax.ShapeDtypeStruct((NWORKERS * NREP, VOCAB_PAD, OUT_DIM),
                                       jnp.float32),
    )(syms_pad, W, b.reshape(1, OUT_DIM))

    idx = QR.reshape(NWORKERS, NCHUNK, CHUNK)
    mesh = plsc.VectorSubcoreMesh(core_axis_name="c", subcore_axis_name="s")
    gather = functools.partial(
        pl.kernel,
        out_type=jax.ShapeDtypeStruct((NTOK, OUT_DIM), jnp.float32),
        mesh=mesh,
        scratch_types=(
            [pltpu.VMEM((NCHUNK, CHUNK), jnp.int32)]
            + [pltpu.VMEM((CHUNK, OUT_DIM), jnp.float32)] * NBUF
            + [pltpu.SemaphoreType.DMA] * (2 * NBUF)
        ),
    )(_gather_body)
    out = gather(table, idx)
    return out.reshape(B, L, OUT_DIM)


# speedup vs baseline: 1.1281x; 1.0115x over previous
import functools

import jax
import jax.numpy as jnp
from jax import lax
from jax.experimental import pallas as pl
from jax.experimental.pallas import tpu as pltpu
from jax.experimental.pallas import tpu_sc as plsc

VOCAB = 73
VOCAB_PAD = 80
GLYPH_DIM = 256
OUT_DIM = 512
B, L = 1024, 200
NTOK = B * L
NWORKERS = 32
PER_W = NTOK // NWORKERS
CHUNK = 32
NCHUNK = PER_W // CHUNK


def _table_body(s_ref, w_ref, b_ref, o_ref):
    o_ref[0] = lax.dot_general(
        s_ref[...], w_ref[...],
        (((1,), (1,)), ((), ())),
        preferred_element_type=jnp.float32,
    ) + b_ref[...]


NBUF = 5
NREP = 1


def _gather_body(table_hbm, idx_hbm, out_hbm, idx_v, *bufs):
    rows = list(bufs[:NBUF])
    gsem = list(bufs[NBUF:2 * NBUF])
    osem = list(bufs[2 * NBUF:])
    cid = lax.axis_index("c")
    sid = lax.axis_index("s")
    wid = sid * 2 + cid
    row_base = wid * PER_W

    pltpu.sync_copy(idx_hbm.at[wid], idx_v)


    def out_slc(c):
        return out_hbm.at[pl.ds(row_base + c * CHUNK, CHUNK)]

    for k in range(NBUF):
        pltpu.async_copy(table_hbm.at[wid].at[idx_v.at[k]], rows[k], gsem[k])

    def step(s, _):
        for k in range(NBUF):
            c = s * NBUF + k
            pltpu.make_async_copy(table_hbm.at[wid].at[idx_v.at[c]],
                                  rows[k], gsem[k]).wait()
            pltpu.async_copy(rows[k], out_slc(c), osem[k])
        for k in range(NBUF):
            c = s * NBUF + k
            pltpu.make_async_copy(rows[k], out_slc(c), osem[k]).wait()

            @pl.when(c + NBUF < NCHUNK)
            def _():
                pltpu.async_copy(table_hbm.at[wid].at[idx_v.at[c + NBUF]],
                                 rows[k], gsem[k])
        return 0

    lax.fori_loop(0, NCHUNK // NBUF, step, 0)


def kernel(QR, syms, W, b):
    syms_pad = jnp.pad(syms, ((0, VOCAB_PAD - VOCAB), (0, 0)))
    table = pl.pallas_call(
        _table_body,
        grid=(NWORKERS,),
        in_specs=[
            pl.BlockSpec((VOCAB_PAD, GLYPH_DIM), lambda i: (0, 0)),
            pl.BlockSpec((OUT_DIM, GLYPH_DIM), lambda i: (0, 0)),
            pl.BlockSpec((1, OUT_DIM), lambda i: (0, 0)),
        ],
        out_specs=pl.BlockSpec((1, VOCAB_PAD, OUT_DIM), lambda i: (i, 0, 0)),
        out_shape=jax.ShapeDtypeStruct((NWORKERS, VOCAB_PAD, OUT_DIM),
                                       jnp.float32),
    )(syms_pad, W, b.reshape(1, OUT_DIM))

    idx = QR.reshape(NWORKERS, NCHUNK, CHUNK)
    mesh = plsc.VectorSubcoreMesh(core_axis_name="c", subcore_axis_name="s")
    gather = functools.partial(
        pl.kernel,
        out_type=jax.ShapeDtypeStruct((NTOK, OUT_DIM), jnp.float32),
        mesh=mesh,
        scratch_types=(
            [pltpu.VMEM((NCHUNK, CHUNK), jnp.int32)]
            + [pltpu.VMEM((CHUNK, OUT_DIM), jnp.float32)] * NBUF
            + [pltpu.SemaphoreType.DMA] * (2 * NBUF)
        ),
    )(_gather_body)
    out = gather(table, idx)
    return out.reshape(B, L, OUT_DIM)
